# baseline (speedup 1.0000x reference)
import jax
import jax.numpy as jnp
from jax import lax
from jax.experimental import pallas as pl
from jax.experimental.pallas import tpu as pltpu
from jax.experimental.pallas import tpu_sc as plsc

_NW = 32
_S = 128
_TC = 32768


def kernel(x, table):
    B, L = x.shape
    V, E = table.shape
    idx = x.astype(jnp.int32)
    tab_t = table.T

    grid_t = (V + _TC - 1) // _TC

    def _widen_body(t_ref, o_ref):
        o_ref[:, :E] = t_ref[...].T

    t128 = pl.pallas_call(
        _widen_body,
        grid=(grid_t,),
        in_specs=[pl.BlockSpec((E, _TC), lambda i: (0, i))],
        out_specs=pl.BlockSpec((_TC, 128), lambda i: (i, 0)),
        out_shape=jax.ShapeDtypeStruct((V, 128), table.dtype),
        compiler_params=pltpu.CompilerParams(dimension_semantics=("parallel",)),
    )(tab_t)

    mesh = plsc.VectorSubcoreMesh(core_axis_name="core", subcore_axis_name="subcore")
    rows_per_worker = B // _NW
    chunks = rows_per_worker // 8

    @pl.kernel(
        out_type=jax.ShapeDtypeStruct((B, L, 128), table.dtype),
        mesh=mesh,
        scratch_types=[
            pltpu.VMEM((8, L), jnp.int32),
            pltpu.VMEM((4, L, 128), jnp.float32),
            pltpu.SemaphoreType.DMA,
            pltpu.SemaphoreType.DMA,
            pltpu.SemaphoreType.DMA,
            pltpu.SemaphoreType.DMA,
            pltpu.SemaphoreType.DMA,
        ],
    )
    def _gather(t_hbm, i_hbm, o_hbm, i_vmem, g_vmem, sem_g0, sem_g1, sem_g2, sem_g3, sem_o):
        w = lax.axis_index("subcore") * 2 + lax.axis_index("core")
        b0 = w * rows_per_worker
        sems = (sem_g0, sem_g1, sem_g2, sem_g3)

        def fire_gathers(r, cb):
            for lo in range(0, L, _S):
                n = min(_S, L - lo)
                pltpu.async_copy(
                    t_hbm.at[i_vmem.at[r, pl.ds(lo, n)]],
                    g_vmem.at[r % 4, pl.ds(lo, n), :],
                    sems[r % 4],
                )

        def wait_gathers(r):
            for lo in range(0, L, _S):
                n = min(_S, L - lo)
                pltpu.make_async_copy(
                    t_hbm.at[i_vmem.at[r, pl.ds(lo, n)]],
                    g_vmem.at[r % 4, pl.ds(lo, n), :],
                    sems[r % 4],
                ).wait()

        def fire_out(r, cb):
            pltpu.async_copy(g_vmem.at[r % 4], o_hbm.at[cb + r], sem_o)

        def wait_out(r, cb):
            pltpu.make_async_copy(
                g_vmem.at[r % 4], o_hbm.at[cb + r], sem_o
            ).wait()

        @pl.loop(0, chunks)
        def _(c):
            cb = b0 + c * 8
            pltpu.sync_copy(i_hbm.at[pl.ds(cb, 8), :], i_vmem)
            fire_gathers(0, cb)
            fire_gathers(1, cb)
            fire_gathers(2, cb)
            for r in range(3, 8):
                if r >= 4:
                    wait_out(r - 4, cb)
                fire_gathers(r, cb)
                wait_gathers(r - 3)
                fire_out(r - 3, cb)
            for r in range(5, 8):
                wait_gathers(r)
                fire_out(r, cb)
            for r in range(4, 8):
                wait_out(r, cb)

    return _gather(t128, idx)[:, :, :E]


# speedup vs baseline: 1.0029x; 1.0029x over previous
import jax
import jax.numpy as jnp
from jax import lax
from jax.experimental import pallas as pl
from jax.experimental.pallas import tpu as pltpu
from jax.experimental.pallas import tpu_sc as plsc

_NW = 32
_S = 128
_TC = 32768


def kernel(x, table):
    B, L = x.shape
    V, E = table.shape
    idx = x.astype(jnp.int32)
    tab_t = table.T

    grid_t = (V + _TC - 1) // _TC

    def _widen_body(t_ref, o_ref):
        o_ref[:, :E] = t_ref[...].T

    t128 = pl.pallas_call(
        _widen_body,
        grid=(grid_t,),
        in_specs=[pl.BlockSpec((E, _TC), lambda i: (0, i))],
        out_specs=pl.BlockSpec((_TC, 128), lambda i: (i, 0)),
        out_shape=jax.ShapeDtypeStruct((V, 128), table.dtype),
        compiler_params=pltpu.CompilerParams(dimension_semantics=("parallel",)),
    )(tab_t)

    mesh = plsc.VectorSubcoreMesh(core_axis_name="core", subcore_axis_name="subcore")
    rows_per_worker = B // _NW
    chunks = rows_per_worker // 8

    @pl.kernel(
        out_type=jax.ShapeDtypeStruct((B, L, 128), table.dtype),
        mesh=mesh,
        scratch_types=[
            pltpu.VMEM((2, 8, L), jnp.int32),
            pltpu.VMEM((4, L, 128), jnp.float32),
            pltpu.SemaphoreType.DMA,
            pltpu.SemaphoreType.DMA,
            pltpu.SemaphoreType.DMA,
            pltpu.SemaphoreType.DMA,
            pltpu.SemaphoreType.DMA,
            pltpu.SemaphoreType.DMA,
        ],
    )
    def _gather(t_hbm, i_hbm, o_hbm, i_vmem, g_vmem, sem_g0, sem_g1, sem_g2, sem_g3, sem_o, sem_i):
        w = lax.axis_index("subcore") * 2 + lax.axis_index("core")
        b0 = w * rows_per_worker
        sems = (sem_g0, sem_g1, sem_g2, sem_g3)

        def idx_copy(c):
            return pltpu.make_async_copy(
                i_hbm.at[pl.ds(b0 + c * 8, 8), :], i_vmem.at[c % 2], sem_i
            )

        def fire_gathers(r, cb, c):
            for lo in range(0, L, _S):
                n = min(_S, L - lo)
                pltpu.async_copy(
                    t_hbm.at[i_vmem.at[c % 2, r, pl.ds(lo, n)]],
                    g_vmem.at[r % 4, pl.ds(lo, n), :],
                    sems[r % 4],
                )

        def wait_gathers(r, c):
            for lo in range(0, L, _S):
                n = min(_S, L - lo)
                pltpu.make_async_copy(
                    t_hbm.at[i_vmem.at[c % 2, r, pl.ds(lo, n)]],
                    g_vmem.at[r % 4, pl.ds(lo, n), :],
                    sems[r % 4],
                ).wait()

        def fire_out(r, cb):
            pltpu.async_copy(g_vmem.at[r % 4], o_hbm.at[cb + r], sem_o)

        def wait_out(r, cb):
            pltpu.make_async_copy(
                g_vmem.at[r % 4], o_hbm.at[cb + r], sem_o
            ).wait()

        idx_copy(0).start()

        @pl.loop(0, chunks)
        def _(c):
            cb = b0 + c * 8
            idx_copy(c).wait()

            @pl.when(c < chunks - 1)
            def _():
                idx_copy(c + 1).start()

            fire_gathers(0, cb, c)
            fire_gathers(1, cb, c)
            fire_gathers(2, cb, c)
            for r in range(3, 8):
                if r >= 4:
                    wait_out(r - 4, cb)
                fire_gathers(r, cb, c)
                wait_gathers(r - 3, c)
                fire_out(r - 3, cb)
            for r in range(5, 8):
                wait_gathers(r, c)
                fire_out(r, cb)
            for r in range(4, 8):
                wait_out(r, cb)

    return _gather(t128, idx)[:, :, :E]
